# baseline (speedup 1.0000x reference)
import functools

import jax
import jax.numpy as jnp
from jax import lax
from jax.experimental import pallas as pl
from jax.experimental.pallas import tpu as pltpu
from jax.experimental.pallas import tpu_sc as plsc

_BITS_07 = 0x3F333333


def _dense_body(x_ref, t_ref, loss_ref, bits_ref, *, nb):
    i = pl.program_id(0)
    x = x_ref[...]
    t = t_ref[...]
    cls = lax.broadcasted_iota(jnp.int32, x.shape, 0)
    e = jnp.exp(x)
    s = jnp.sum(e, axis=0)
    tx = jnp.sum(jnp.where(cls == t[None], x, 0.0), axis=0)
    loss = jnp.log(s) - tx
    pred = jnp.exp(-loss)
    loss_ref[...] = loss
    bits_ref[...] = lax.bitcast_convert_type(pred, jnp.int32)


def _search_body(bits_ref, loss_ref, out_ref, *, kth):
    bits = bits_ref[...]

    def bs_body(_, carry):
        lo, hi = carry
        mid = lax.div(lo + hi, 2)
        cnt = jnp.sum((bits <= mid).astype(jnp.int32))
        geq = cnt >= kth + 1
        return (jnp.where(geq, lo, mid + 1), jnp.where(geq, mid, hi))

    lo, _ = lax.fori_loop(0, 31, bs_body, (jnp.int32(0), jnp.int32(1 << 30)))
    thr = jnp.maximum(lo, _BITS_07)
    keep = bits < thr
    num = jnp.sum(jnp.where(keep, loss_ref[...], 0.0))
    den = jnp.sum(keep.astype(jnp.float32))
    out_ref[...] = (num / den)[None, None]


def _sc_partials(bits_flat, loss_flat, *, chunk):
    mesh = plsc.VectorSubcoreMesh(core_axis_name="c", subcore_axis_name="s")

    @functools.partial(
        pl.kernel, mesh=mesh,
        out_type=jax.ShapeDtypeStruct((64, 16), jnp.float32),
        scratch_types=[
            pltpu.VMEM((chunk,), jnp.int32),
            pltpu.VMEM((chunk,), jnp.float32),
            pltpu.VMEM((2, 16), jnp.float32),
        ],
    )
    def sc_kernel(bits_hbm, loss_hbm, out_hbm, bits_v, loss_v, res_v):
        wid = lax.axis_index("s") * 2 + lax.axis_index("c")
        base = wid * chunk
        pltpu.sync_copy(bits_hbm.at[pl.ds(base, chunk)], bits_v)
        pltpu.sync_copy(loss_hbm.at[pl.ds(base, chunk)], loss_v)

        def body(j, carry):
            cnt_v, sm_v = carry
            b = bits_v[pl.ds(j * 16, 16)]
            l = loss_v[pl.ds(j * 16, 16)]
            kf = jnp.where(b < _BITS_07, 1.0, 0.0)
            return cnt_v + kf, sm_v + l * kf

        zero = jnp.zeros((16,), jnp.float32)
        cnt_v, sm_v = lax.fori_loop(0, chunk // 16, body, (zero, zero))
        res_v[0] = cnt_v
        res_v[1] = sm_v
        pltpu.sync_copy(res_v, out_hbm.at[pl.ds(2 * wid, 2)])

    return sc_kernel(bits_flat, loss_flat)


def kernel(score, target):
    n, c = score.shape
    lanes = 128
    srows = n // lanes
    bs = 256
    nb = srows // bs
    kth = min(int(0.7 * n), n - 1)

    xt3 = jnp.transpose(score.reshape(srows, lanes, c), (2, 0, 1))
    t2 = target.reshape(srows, lanes)

    loss2d, bits2d = pl.pallas_call(
        functools.partial(_dense_body, nb=nb),
        grid=(nb,),
        in_specs=[
            pl.BlockSpec((c, bs, lanes), lambda i: (0, i, 0)),
            pl.BlockSpec((bs, lanes), lambda i: (i, 0)),
        ],
        out_specs=[
            pl.BlockSpec((bs, lanes), lambda i: (i, 0)),
            pl.BlockSpec((bs, lanes), lambda i: (i, 0)),
        ],
        out_shape=[
            jax.ShapeDtypeStruct((srows, lanes), jnp.float32),
            jax.ShapeDtypeStruct((srows, lanes), jnp.int32),
        ],
    )(xt3, t2)

    partials = _sc_partials(bits2d.reshape(-1), loss2d.reshape(-1),
                            chunk=n // 32)
    cnt = jnp.sum(partials[0::2])
    sm = jnp.sum(partials[1::2])

    def _fast(_):
        return sm / cnt

    def _search(_):
        out = pl.pallas_call(
            functools.partial(_search_body, kth=kth),
            out_shape=jax.ShapeDtypeStruct((1, 1), jnp.float32),
        )(bits2d, loss2d)
        return out[0, 0]

    return lax.cond(cnt > jnp.float32(kth), _fast, _search, None)
